# baseline (speedup 1.0000x reference)
import functools

import jax
import jax.numpy as jnp
from jax import lax
from jax.experimental import pallas as pl
from jax.experimental.pallas import tpu as pltpu
from jax.experimental.pallas import tpu_sc as plsc

B = 16384
EMB = 32
EPS = 1e-3
NC = 2
NS = 16
NW = NC * NS
BPW = B // NW

CHUNK = 16

BLK = 2048


def _sc_gather_body(user_hbm, op_hbm, cid_hbm, oid_hbm, ce_hbm, oe_hbm,
                    idx_u_v, idx_o_v, sem_u, sem_o):
    wid = lax.axis_index("s") * NC + lax.axis_index("c")
    base = wid * BPW
    pltpu.sync_copy(cid_hbm.at[pl.ds(base, BPW)], idx_u_v)
    pltpu.sync_copy(oid_hbm.at[pl.ds(base, BPW)], idx_o_v)

    def chunk(c, _):
        i0 = c * CHUNK
        vec_u = idx_u_v[pl.ds(i0, CHUNK)]
        vec_o = idx_o_v[pl.ds(i0, CHUNK)]
        cps = []
        for j in range(CHUNK):
            cps.append(pltpu.async_copy(
                user_hbm.at[pl.ds(vec_u[j], 1)],
                ce_hbm.at[pl.ds(base + i0 + j, 1)], sem_u))
            cps.append(pltpu.async_copy(
                op_hbm.at[pl.ds(vec_o[j], 1)],
                oe_hbm.at[pl.ds(base + i0 + j, 1)], sem_o))
        for cp in cps:
            cp.wait()
        return _

    lax.fori_loop(0, BPW // CHUNK, chunk, 0)


def _sc_gather(user_table, op_table, customer_id, operator_name):
    mesh = plsc.VectorSubcoreMesh(core_axis_name="c", subcore_axis_name="s",
                                  num_cores=NC, num_subcores=NS)
    return pl.kernel(
        _sc_gather_body,
        out_type=(jax.ShapeDtypeStruct((B, EMB), jnp.float32),
                  jax.ShapeDtypeStruct((B, EMB), jnp.float32)),
        mesh=mesh,
        scratch_types=[
            pltpu.VMEM((BPW,), jnp.int32),
            pltpu.VMEM((BPW,), jnp.int32),
            pltpu.SemaphoreType.DMA,
            pltpu.SemaphoreType.DMA,
        ],
        compiler_params=pltpu.CompilerParams(use_tc_tiling_on_sc=True),
    )(user_table, op_table, customer_id, operator_name)


def _mlp_body(ce_ref, oe_ref, W1_ref, b1_ref, g1_ref, be1_ref, m1_ref, v1_ref,
              W2_ref, b2_ref, g2_ref, be2_ref, m2_ref, v2_ref,
              W3_ref, b3_ref, out_ref):
    s1 = g1_ref[...] * lax.rsqrt(v1_ref[...] + EPS)
    c1 = (b1_ref[...] - m1_ref[...]) * s1 + be1_ref[...]
    s2 = g2_ref[...] * lax.rsqrt(v2_ref[...] + EPS)
    c2 = (b2_ref[...] - m2_ref[...]) * s2 + be2_ref[...]

    W1 = W1_ref[...] * s1
    acc = jnp.dot(ce_ref[...], W1[:EMB, :],
                  preferred_element_type=jnp.float32)
    acc += jnp.dot(oe_ref[...], W1[EMB:, :],
                   preferred_element_type=jnp.float32)
    h1 = jnp.maximum(acc + c1, 0.0)

    W2 = W2_ref[...] * s2
    h2 = jnp.maximum(jnp.dot(h1, W2, preferred_element_type=jnp.float32) + c2,
                     0.0)

    out_ref[...] = (jnp.dot(h2, W3_ref[...],
                            preferred_element_type=jnp.float32)
                    + b3_ref[...])


def _mlp(ce, oe, W1, b1, g1, be1, m1, v1, W2, b2, g2, be2, m2, v2, W3, b3):
    grid = (B // BLK,)
    full = lambda shape: pl.BlockSpec(shape, lambda i: (0, 0))
    return pl.pallas_call(
        _mlp_body,
        grid=grid,
        in_specs=[
            pl.BlockSpec((BLK, EMB), lambda i: (i, 0)),
            pl.BlockSpec((BLK, EMB), lambda i: (i, 0)),
            full((2 * EMB, 256)), full((1, 256)), full((1, 256)),
            full((1, 256)), full((1, 256)), full((1, 256)),
            full((256, 128)), full((1, 128)), full((1, 128)),
            full((1, 128)), full((1, 128)), full((1, 128)),
            full((128, 1)), full((1, 1)),
        ],
        out_specs=pl.BlockSpec((BLK, 1), lambda i: (i, 0)),
        out_shape=jax.ShapeDtypeStruct((B, 1), jnp.float32),
    )(ce, oe, W1, b1.reshape(1, -1), g1.reshape(1, -1), be1.reshape(1, -1),
      m1.reshape(1, -1), v1.reshape(1, -1), W2, b2.reshape(1, -1),
      g2.reshape(1, -1), be2.reshape(1, -1), m2.reshape(1, -1),
      v2.reshape(1, -1), W3, b3.reshape(1, -1))


def kernel(customer_id, operator_name, user_table, op_table,
           W1, b1, g1, be1, m1, v1, W2, b2, g2, be2, m2, v2, W3, b3):
    ce, oe = _sc_gather(user_table, op_table,
                        customer_id.astype(jnp.int32),
                        operator_name.astype(jnp.int32))
    return _mlp(ce, oe, W1, b1, g1, be1, m1, v1,
                W2, b2, g2, be2, m2, v2, W3, b3)


# speedup vs baseline: 4.7252x; 4.7252x over previous
import functools

import jax
import jax.numpy as jnp
from jax import lax
from jax.experimental import pallas as pl
from jax.experimental.pallas import tpu as pltpu
from jax.experimental.pallas import tpu_sc as plsc

B = 16384
EMB = 32
EPS = 1e-3
NC = 2
NS = 16
NW = NC * NS
BPW = B // NW

CHUNK = 16

BLK = 2048
BLK4 = BLK // 4


def _sc_gather_body(user_hbm, op_hbm, cid_hbm, oid_hbm, ce_hbm, oe_hbm,
                    idx_u_v, idx_o_v, rows, sem):
    wid = lax.axis_index("s") * NC + lax.axis_index("c")
    base = wid * BPW
    pltpu.sync_copy(cid_hbm.at[pl.ds(base, BPW)], idx_u_v)
    pltpu.sync_copy(oid_hbm.at[pl.ds(base, BPW)], idx_o_v)

    def make_pass(table_hbm, idx_v, out_hbm):
        def chunk(c, _):
            i0 = c * CHUNK
            vec = idx_v[pl.ds(i0, CHUNK)]
            cps = []
            for j in range(CHUNK):
                cps.append(pltpu.async_copy(
                    table_hbm.at[pl.ds(vec[j], 1)],
                    rows.at[pl.ds(i0 + j, 1)], sem))
            for cp in cps:
                cp.wait()
            return _

        lax.fori_loop(0, BPW // CHUNK, chunk, 0)
        pltpu.sync_copy(rows, out_hbm.at[pl.ds(base, BPW)])

    make_pass(user_hbm, idx_u_v, ce_hbm)
    make_pass(op_hbm, idx_o_v, oe_hbm)


def _sc_gather(user_table, op_table, customer_id, operator_name):
    mesh = plsc.VectorSubcoreMesh(core_axis_name="c", subcore_axis_name="s",
                                  num_cores=NC, num_subcores=NS)
    return pl.kernel(
        _sc_gather_body,
        out_type=(jax.ShapeDtypeStruct((B, EMB), jnp.float32),
                  jax.ShapeDtypeStruct((B, EMB), jnp.float32)),
        mesh=mesh,
        scratch_types=[
            pltpu.VMEM((BPW,), jnp.int32),
            pltpu.VMEM((BPW,), jnp.int32),
            pltpu.VMEM((BPW, EMB), jnp.float32),
            pltpu.SemaphoreType.DMA,
        ],
        compiler_params=pltpu.CompilerParams(use_tc_tiling_on_sc=True),
    )(user_table, op_table, customer_id, operator_name)


def _mlp_body(ce_ref, oe_ref, W1_ref, b1_ref, g1_ref, be1_ref, m1_ref, v1_ref,
              W2_ref, b2_ref, g2_ref, be2_ref, m2_ref, v2_ref,
              W3_ref, b3_ref, out_ref):
    s1 = g1_ref[...] * lax.rsqrt(v1_ref[...] + EPS)
    c1 = (b1_ref[...] - m1_ref[...]) * s1 + be1_ref[...]
    s2 = g2_ref[...] * lax.rsqrt(v2_ref[...] + EPS)
    c2 = (b2_ref[...] - m2_ref[...]) * s2 + be2_ref[...]

    W1 = W1_ref[...] * s1
    acc = jnp.dot(ce_ref[...], W1[:EMB, :],
                  preferred_element_type=jnp.float32)
    acc += jnp.dot(oe_ref[...], W1[EMB:, :],
                   preferred_element_type=jnp.float32)
    h1 = jnp.maximum(acc + c1, 0.0)

    W2 = W2_ref[...] * s2
    h2 = jnp.maximum(jnp.dot(h1, W2, preferred_element_type=jnp.float32) + c2,
                     0.0)

    out_ref[...] = (jnp.dot(h2, W3_ref[...],
                            preferred_element_type=jnp.float32)
                    + b3_ref[...])


def _mlp(ce, oe, W1, b1, g1, be1, m1, v1, W2, b2, g2, be2, m2, v2, W3, b3):
    grid = (B // BLK,)
    full = lambda shape: pl.BlockSpec(shape, lambda i: (0, 0))
    return pl.pallas_call(
        _mlp_body,
        grid=grid,
        in_specs=[
            pl.BlockSpec((BLK, EMB), lambda i: (i, 0)),
            pl.BlockSpec((BLK, EMB), lambda i: (i, 0)),
            full((2 * EMB, 256)), full((1, 256)), full((1, 256)),
            full((1, 256)), full((1, 256)), full((1, 256)),
            full((256, 128)), full((1, 128)), full((1, 128)),
            full((1, 128)), full((1, 128)), full((1, 128)),
            full((128, 1)), full((1, 1)),
        ],
        out_specs=pl.BlockSpec((BLK, 1), lambda i: (i, 0)),
        out_shape=jax.ShapeDtypeStruct((B, 1), jnp.float32),
    )(ce, oe, W1, b1.reshape(1, -1), g1.reshape(1, -1), be1.reshape(1, -1),
      m1.reshape(1, -1), v1.reshape(1, -1), W2, b2.reshape(1, -1),
      g2.reshape(1, -1), be2.reshape(1, -1), m2.reshape(1, -1),
      v2.reshape(1, -1), W3, b3.reshape(1, -1))


def kernel(customer_id, operator_name, user_table, op_table,
           W1, b1, g1, be1, m1, v1, W2, b2, g2, be2, m2, v2, W3, b3):
    ce, oe = _sc_gather(user_table, op_table,
                        customer_id.astype(jnp.int32),
                        operator_name.astype(jnp.int32))
    return _mlp(ce, oe, W1, b1, g1, be1, m1, v1,
                W2, b2, g2, be2, m2, v2, W3, b3)


# speedup vs baseline: 5.4479x; 1.1529x over previous
import functools

import jax
import jax.numpy as jnp
from jax import lax
from jax.experimental import pallas as pl
from jax.experimental.pallas import tpu as pltpu
from jax.experimental.pallas import tpu_sc as plsc

B = 16384
EMB = 32
EPS = 1e-3
NC = 2
NS = 16
NW = NC * NS
BPW = B // NW

CHUNK = 16
NCHUNK = BPW // CHUNK

BLK = 2048
BLK4 = BLK // 4


def _sc_gather_body(user_hbm, op_hbm, cid_hbm, oid_hbm, ce_hbm, oe_hbm,
                    idx_u_v, idx_o_v, rows_u, rows_o, sem):
    wid = lax.axis_index("s") * NC + lax.axis_index("c")
    base = wid * BPW
    pltpu.sync_copy(cid_hbm.at[pl.ds(base, BPW)], idx_u_v)
    pltpu.sync_copy(oid_hbm.at[pl.ds(base, BPW)], idx_o_v)

    def issue(c):
        i0 = c * CHUNK
        vec_u = idx_u_v[pl.ds(i0, CHUNK)]
        vec_o = idx_o_v[pl.ds(i0, CHUNK)]
        for j in range(CHUNK):
            o = (i0 + j) * EMB
            pltpu.async_copy(user_hbm.at[pl.ds(vec_u[j] * EMB, EMB)],
                             rows_u.at[pl.ds(o, EMB)], sem)
            pltpu.async_copy(op_hbm.at[pl.ds(vec_o[j] * EMB, EMB)],
                             rows_o.at[pl.ds(o, EMB)], sem)

    def drain_one_chunk():
        for j in range(2 * CHUNK):
            pltpu.make_async_copy(user_hbm.at[pl.ds(0, EMB)],
                                  rows_u.at[pl.ds(j * EMB, EMB)], sem).wait()

    def step(c, carry):
        issue(c)

        @pl.when(c > 0)
        def _():
            drain_one_chunk()
        return carry

    lax.fori_loop(0, NCHUNK, step, 0)
    drain_one_chunk()
    pltpu.sync_copy(rows_u, ce_hbm.at[pl.ds(base * EMB, BPW * EMB)])
    pltpu.sync_copy(rows_o, oe_hbm.at[pl.ds(base * EMB, BPW * EMB)])


def _sc_gather(user_flat, op_flat, customer_id, operator_name):
    mesh = plsc.VectorSubcoreMesh(core_axis_name="c", subcore_axis_name="s",
                                  num_cores=NC, num_subcores=NS)
    return pl.kernel(
        _sc_gather_body,
        out_type=(jax.ShapeDtypeStruct((B * EMB,), jnp.float32),
                  jax.ShapeDtypeStruct((B * EMB,), jnp.float32)),
        mesh=mesh,
        scratch_types=[
            pltpu.VMEM((BPW,), jnp.int32),
            pltpu.VMEM((BPW,), jnp.int32),
            pltpu.VMEM((BPW * EMB,), jnp.float32),
            pltpu.VMEM((BPW * EMB,), jnp.float32),
            pltpu.SemaphoreType.DMA,
        ],
        compiler_params=pltpu.CompilerParams(use_tc_tiling_on_sc=False),
    )(user_flat, op_flat, customer_id, operator_name)


def _mlp_body(ce_ref, oe_ref, W1_ref, b1_ref, g1_ref, be1_ref, m1_ref, v1_ref,
              W2_ref, b2_ref, g2_ref, be2_ref, m2_ref, v2_ref,
              W3_ref, b3_ref, out_ref):
    s1 = g1_ref[...] * lax.rsqrt(v1_ref[...] + EPS)
    c1 = (b1_ref[...] - m1_ref[...]) * s1 + be1_ref[...]
    s2 = g2_ref[...] * lax.rsqrt(v2_ref[...] + EPS)
    c2 = (b2_ref[...] - m2_ref[...]) * s2 + be2_ref[...]

    W1 = W1_ref[...] * s1
    W2 = W2_ref[...] * s2
    W3 = W3_ref[...]

    cep = ce_ref[...]
    oep = oe_ref[...]
    outs = []
    for j in range(4):
        ce = cep[:, j * EMB:(j + 1) * EMB]
        oe = oep[:, j * EMB:(j + 1) * EMB]
        acc = jnp.dot(ce, W1[:EMB, :], preferred_element_type=jnp.float32)
        acc += jnp.dot(oe, W1[EMB:, :], preferred_element_type=jnp.float32)
        h1 = jnp.maximum(acc + c1, 0.0)
        h2 = jnp.maximum(
            jnp.dot(h1, W2, preferred_element_type=jnp.float32) + c2, 0.0)
        outs.append(jnp.dot(h2, W3, preferred_element_type=jnp.float32))
    out_ref[...] = jnp.concatenate(outs, axis=1) + b3_ref[...]


def _mlp(cep, oep, W1, b1, g1, be1, m1, v1, W2, b2, g2, be2, m2, v2, W3, b3):
    grid = (B // BLK,)
    full = lambda shape: pl.BlockSpec(shape, lambda i: (0, 0))
    return pl.pallas_call(
        _mlp_body,
        grid=grid,
        in_specs=[
            pl.BlockSpec((BLK4, 128), lambda i: (i, 0)),
            pl.BlockSpec((BLK4, 128), lambda i: (i, 0)),
            full((2 * EMB, 256)), full((1, 256)), full((1, 256)),
            full((1, 256)), full((1, 256)), full((1, 256)),
            full((256, 128)), full((1, 128)), full((1, 128)),
            full((1, 128)), full((1, 128)), full((1, 128)),
            full((128, 1)), full((1, 1)),
        ],
        out_specs=pl.BlockSpec((BLK4, 4), lambda i: (i, 0)),
        out_shape=jax.ShapeDtypeStruct((B // 4, 4), jnp.float32),
    )(cep, oep, W1, b1.reshape(1, -1), g1.reshape(1, -1), be1.reshape(1, -1),
      m1.reshape(1, -1), v1.reshape(1, -1), W2, b2.reshape(1, -1),
      g2.reshape(1, -1), be2.reshape(1, -1), m2.reshape(1, -1),
      v2.reshape(1, -1), W3, b3.reshape(1, -1))


def kernel(customer_id, operator_name, user_table, op_table,
           W1, b1, g1, be1, m1, v1, W2, b2, g2, be2, m2, v2, W3, b3):
    ce_flat, oe_flat = _sc_gather(user_table.reshape(-1),
                                  op_table.reshape(-1),
                                  customer_id.astype(jnp.int32),
                                  operator_name.astype(jnp.int32))
    cep = ce_flat.reshape(B // 4, 128)
    oep = oe_flat.reshape(B // 4, 128)
    out4 = _mlp(cep, oep, W1, b1, g1, be1, m1, v1,
                W2, b2, g2, be2, m2, v2, W3, b3)
    return out4.reshape(B, 1)


# speedup vs baseline: 6.5466x; 1.2017x over previous
import functools

import jax
import jax.numpy as jnp
from jax import lax
from jax.experimental import pallas as pl
from jax.experimental.pallas import tpu as pltpu
from jax.experimental.pallas import tpu_sc as plsc

B = 16384
EMB = 32
EPS = 1e-3
NC = 2
NS = 16
FPS = EMB // NC
FPP = FPS // 2
BPT = B // NS
UROW = 100016
HROW = UROW // 2
OROW = 1008
ICH = 128
NCH = BPT // ICH

BLKC = 2048


def _sc_gather_body(ut_hbm, ot_hbm, cid_hbm, oid_hbm, ce_hbm, oe_hbm,
                    idx_u, idx_o, dst_u, dst_o, ushr, oshr, sem, gsem):
    s = lax.axis_index("c")
    t = lax.axis_index("s")
    col0 = t * BPT

    pltpu.sync_copy(cid_hbm.at[pl.ds(col0, BPT)], idx_u)
    pltpu.sync_copy(oid_hbm.at[pl.ds(col0, BPT)], idx_o)
    pltpu.async_copy(ot_hbm.at[s * FPS + t], oshr.at[t], sem).wait()

    for p in range(2):
        fl_stage = t // 2
        half = t % 2
        pltpu.async_copy(
            ut_hbm.at[s * FPS + p * FPP + fl_stage, pl.ds(half * HROW, HROW)],
            ushr.at[fl_stage, pl.ds(half * HROW, HROW)], sem).wait()
        plsc.subcore_barrier()

        for fl in range(FPP):
            cps = []
            for k in range(NCH):
                cps.append(pltpu.async_copy(
                    ushr.at[fl].at[idx_u.at[pl.ds(k * ICH, ICH)]],
                    dst_u.at[fl, pl.ds(k * ICH, ICH)], gsem))
                cps.append(pltpu.async_copy(
                    oshr.at[p * FPP + fl].at[idx_o.at[pl.ds(k * ICH, ICH)]],
                    dst_o.at[fl, pl.ds(k * ICH, ICH)], gsem))
            for cp in cps:
                cp.wait()

        row0 = s * FPS + p * FPP
        pltpu.sync_copy(dst_u, ce_hbm.at[pl.ds(row0, FPP), pl.ds(col0, BPT)])
        pltpu.sync_copy(dst_o, oe_hbm.at[pl.ds(row0, FPP), pl.ds(col0, BPT)])
        plsc.subcore_barrier()


def _sc_gather(ut, ot, customer_id, operator_name):
    mesh = plsc.VectorSubcoreMesh(core_axis_name="c", subcore_axis_name="s",
                                  num_cores=NC, num_subcores=NS)
    return pl.kernel(
        _sc_gather_body,
        out_type=(jax.ShapeDtypeStruct((EMB, B), jnp.float32),
                  jax.ShapeDtypeStruct((EMB, B), jnp.float32)),
        mesh=mesh,
        scratch_types=[
            pltpu.VMEM((BPT,), jnp.int32),
            pltpu.VMEM((BPT,), jnp.int32),
            pltpu.VMEM((FPP, BPT), jnp.float32),
            pltpu.VMEM((FPP, BPT), jnp.float32),
            pltpu.VMEM_SHARED((FPP, UROW), jnp.float32),
            pltpu.VMEM_SHARED((FPS, OROW), jnp.float32),
            pltpu.SemaphoreType.DMA,
            pltpu.SemaphoreType.DMA,
        ],
        compiler_params=pltpu.CompilerParams(use_tc_tiling_on_sc=False),
    )(ut, ot, customer_id, operator_name)


def _mlp_body(xc_ref, xo_ref, W1_ref, b1_ref, g1_ref, be1_ref, m1_ref, v1_ref,
              W2_ref, b2_ref, g2_ref, be2_ref, m2_ref, v2_ref,
              W3_ref, b3_ref, out_ref):
    s1 = g1_ref[...] * lax.rsqrt(v1_ref[...] + EPS)
    c1 = (b1_ref[...] - m1_ref[...]) * s1 + be1_ref[...]
    s2 = g2_ref[...] * lax.rsqrt(v2_ref[...] + EPS)
    c2 = (b2_ref[...] - m2_ref[...]) * s2 + be2_ref[...]

    W1 = W1_ref[...]
    cn = (((0,), (0,)), ((), ()))
    acc = lax.dot_general(W1[:EMB, :], xc_ref[...], cn,
                          preferred_element_type=jnp.float32)
    acc += lax.dot_general(W1[EMB:, :], xo_ref[...], cn,
                           preferred_element_type=jnp.float32)
    h1 = jnp.maximum(acc * s1 + c1, 0.0)
    h2 = jnp.maximum(
        lax.dot_general(W2_ref[...], h1, cn,
                        preferred_element_type=jnp.float32) * s2 + c2, 0.0)
    out_ref[...] = (lax.dot_general(W3_ref[...], h2, cn,
                                    preferred_element_type=jnp.float32)
                    + b3_ref[...])


def _mlp(xc, xo, W1, b1, g1, be1, m1, v1, W2, b2, g2, be2, m2, v2, W3, b3):
    grid = (B // BLKC,)
    full = lambda shape: pl.BlockSpec(shape, lambda i: (0, 0))
    return pl.pallas_call(
        _mlp_body,
        grid=grid,
        in_specs=[
            pl.BlockSpec((EMB, BLKC), lambda i: (0, i)),
            pl.BlockSpec((EMB, BLKC), lambda i: (0, i)),
            full((2 * EMB, 256)), full((256, 1)), full((256, 1)),
            full((256, 1)), full((256, 1)), full((256, 1)),
            full((256, 128)), full((128, 1)), full((128, 1)),
            full((128, 1)), full((128, 1)), full((128, 1)),
            full((128, 1)), full((1, 1)),
        ],
        out_specs=pl.BlockSpec((1, BLKC), lambda i: (0, i)),
        out_shape=jax.ShapeDtypeStruct((1, B), jnp.float32),
    )(xc, xo, W1, b1.reshape(-1, 1), g1.reshape(-1, 1), be1.reshape(-1, 1),
      m1.reshape(-1, 1), v1.reshape(-1, 1), W2, b2.reshape(-1, 1),
      g2.reshape(-1, 1), be2.reshape(-1, 1), m2.reshape(-1, 1),
      v2.reshape(-1, 1), W3, b3.reshape(1, 1))


def kernel(customer_id, operator_name, user_table, op_table,
           W1, b1, g1, be1, m1, v1, W2, b2, g2, be2, m2, v2, W3, b3):
    ut = jnp.pad(user_table.T, ((0, 0), (0, UROW - user_table.shape[0])))
    ot = jnp.pad(op_table.T, ((0, 0), (0, OROW - op_table.shape[0])))
    ce_t, oe_t = _sc_gather(ut, ot,
                            customer_id.astype(jnp.int32),
                            operator_name.astype(jnp.int32))
    out = _mlp(ce_t, oe_t, W1, b1, g1, be1, m1, v1,
               W2, b2, g2, be2, m2, v2, W3, b3)
    return out.reshape(B, 1)


# speedup vs baseline: 6.6759x; 1.0198x over previous
import functools

import jax
import jax.numpy as jnp
from jax import lax
from jax.experimental import pallas as pl
from jax.experimental.pallas import tpu as pltpu
from jax.experimental.pallas import tpu_sc as plsc

B = 16384
EMB = 32
EPS = 1e-3
NC = 2
NS = 16
FPS = EMB // NC
FPP = FPS // 2
BPT = B // NS
UROW = 100096
HROW = UROW // 2
UCH = 4352
OROW = 1008
ICH = 128
NCH = BPT // ICH

BLKC = 2048


def _sc_gather_body(ut_hbm, ot_hbm, cid_hbm, oid_hbm, ce_hbm, oe_hbm,
                    idx_u, idx_o, dst_u, dst_o, ushr, oshr, sem, gsem):
    s = lax.axis_index("c")
    t = lax.axis_index("s")
    col0 = t * BPT

    pltpu.sync_copy(cid_hbm.at[pl.ds(col0, BPT)], idx_u)
    pltpu.sync_copy(oid_hbm.at[pl.ds(col0, BPT)], idx_o)
    pltpu.async_copy(ot_hbm.at[s * FPS + t], oshr.at[t], sem).wait()

    for p in range(2):
        fl_stage = t // 2
        half = t % 2
        pltpu.async_copy(
            ut_hbm.at[s * FPS + p * FPP + fl_stage, pl.ds(half * HROW, HROW)],
            ushr.at[fl_stage, pl.ds(half * HROW, HROW)], sem).wait()
        plsc.subcore_barrier()

        for fl in range(FPP):
            cps = []
            for k in range(NCH):
                cps.append(pltpu.async_copy(
                    ushr.at[fl].at[idx_u.at[pl.ds(k * ICH, ICH)]],
                    dst_u.at[fl, pl.ds(k * ICH, ICH)], gsem))
                cps.append(pltpu.async_copy(
                    oshr.at[p * FPP + fl].at[idx_o.at[pl.ds(k * ICH, ICH)]],
                    dst_o.at[fl, pl.ds(k * ICH, ICH)], gsem))
            for cp in cps:
                cp.wait()

        row0 = s * FPS + p * FPP
        pltpu.sync_copy(dst_u, ce_hbm.at[pl.ds(row0, FPP), pl.ds(col0, BPT)])
        pltpu.sync_copy(dst_o, oe_hbm.at[pl.ds(row0, FPP), pl.ds(col0, BPT)])
        plsc.subcore_barrier()


def _sc_gather(ut, ot, customer_id, operator_name):
    mesh = plsc.VectorSubcoreMesh(core_axis_name="c", subcore_axis_name="s",
                                  num_cores=NC, num_subcores=NS)
    return pl.kernel(
        _sc_gather_body,
        out_type=(jax.ShapeDtypeStruct((EMB, B), jnp.float32),
                  jax.ShapeDtypeStruct((EMB, B), jnp.float32)),
        mesh=mesh,
        scratch_types=[
            pltpu.VMEM((BPT,), jnp.int32),
            pltpu.VMEM((BPT,), jnp.int32),
            pltpu.VMEM((FPP, BPT), jnp.float32),
            pltpu.VMEM((FPP, BPT), jnp.float32),
            pltpu.VMEM_SHARED((FPP, UROW), jnp.float32),
            pltpu.VMEM_SHARED((FPS, OROW), jnp.float32),
            pltpu.SemaphoreType.DMA,
            pltpu.SemaphoreType.DMA,
        ],
        compiler_params=pltpu.CompilerParams(use_tc_tiling_on_sc=False),
    )(ut, ot, customer_id, operator_name)


def _repack_body(in_ref, out_ref):
    out_ref[...] = in_ref[...]


def _repack(ut):
    return pl.pallas_call(
        _repack_body,
        grid=(UROW // UCH,),
        in_specs=[pl.BlockSpec((EMB, UCH), lambda i: (0, i))],
        out_specs=pl.BlockSpec((EMB, UCH), lambda i: (0, i)),
        out_shape=jax.ShapeDtypeStruct((EMB, UROW), jnp.float32),
    )(ut)


def _mlp_body(xc_ref, xo_ref, W1_ref, b1_ref, g1_ref, be1_ref, m1_ref, v1_ref,
              W2_ref, b2_ref, g2_ref, be2_ref, m2_ref, v2_ref,
              W3_ref, b3_ref, out_ref):
    s1 = g1_ref[...] * lax.rsqrt(v1_ref[...] + EPS)
    c1 = (b1_ref[...] - m1_ref[...]) * s1 + be1_ref[...]
    s2 = g2_ref[...] * lax.rsqrt(v2_ref[...] + EPS)
    c2 = (b2_ref[...] - m2_ref[...]) * s2 + be2_ref[...]

    W1 = W1_ref[...]
    cn = (((0,), (0,)), ((), ()))
    acc = lax.dot_general(W1[:EMB, :], xc_ref[...], cn,
                          preferred_element_type=jnp.float32)
    acc += lax.dot_general(W1[EMB:, :], xo_ref[...], cn,
                           preferred_element_type=jnp.float32)
    h1 = jnp.maximum(acc * s1 + c1, 0.0)
    h2 = jnp.maximum(
        lax.dot_general(W2_ref[...], h1, cn,
                        preferred_element_type=jnp.float32) * s2 + c2, 0.0)
    out_ref[...] = (lax.dot_general(W3_ref[...], h2, cn,
                                    preferred_element_type=jnp.float32)
                    + b3_ref[...])


def _mlp(xc, xo, W1, b1, g1, be1, m1, v1, W2, b2, g2, be2, m2, v2, W3, b3):
    grid = (B // BLKC,)
    full = lambda shape: pl.BlockSpec(shape, lambda i: (0, 0))
    return pl.pallas_call(
        _mlp_body,
        grid=grid,
        in_specs=[
            pl.BlockSpec((EMB, BLKC), lambda i: (0, i)),
            pl.BlockSpec((EMB, BLKC), lambda i: (0, i)),
            full((2 * EMB, 256)), full((256, 1)), full((256, 1)),
            full((256, 1)), full((256, 1)), full((256, 1)),
            full((256, 128)), full((128, 1)), full((128, 1)),
            full((128, 1)), full((128, 1)), full((128, 1)),
            full((128, 1)), full((1, 1)),
        ],
        out_specs=pl.BlockSpec((1, BLKC), lambda i: (0, i)),
        out_shape=jax.ShapeDtypeStruct((1, B), jnp.float32),
    )(xc, xo, W1, b1.reshape(-1, 1), g1.reshape(-1, 1), be1.reshape(-1, 1),
      m1.reshape(-1, 1), v1.reshape(-1, 1), W2, b2.reshape(-1, 1),
      g2.reshape(-1, 1), be2.reshape(-1, 1), m2.reshape(-1, 1),
      v2.reshape(-1, 1), W3, b3.reshape(1, 1))


def kernel(customer_id, operator_name, user_table, op_table,
           W1, b1, g1, be1, m1, v1, W2, b2, g2, be2, m2, v2, W3, b3):
    ut = _repack(user_table.T)
    ot = jnp.pad(op_table.T, ((0, 0), (0, OROW - op_table.shape[0])))
    ce_t, oe_t = _sc_gather(ut, ot,
                            customer_id.astype(jnp.int32),
                            operator_name.astype(jnp.int32))
    out = _mlp(ce_t, oe_t, W1, b1, g1, be1, m1, v1,
               W2, b2, g2, be2, m2, v2, W3, b3)
    return out.reshape(B, 1)
